# Initial kernel scaffold; baseline (speedup 1.0000x reference)
import jax
import jax.numpy as jnp
from jax.experimental import pallas as pl


def kernel(x, edges, kernel, kernel_attention1, kernel_attention2, bias):
    raise NotImplementedError("write your pallas kernel here")


# speedup vs baseline: 48.4265x; 48.4265x over previous
import functools

import jax
import jax.numpy as jnp
from jax import lax
from jax.experimental import pallas as pl
from jax.experimental.pallas import tpu as pltpu
from jax.experimental.pallas import tpu_sc as plsc

H = 8
U = 16
NC = 2
NS = 16
LANES = 16
D = H * U


def _pre_body(x_ref, w_ref, a1_ref, a2_ref, xp_ref, fboth_ref):
    x = x_ref[...]
    w = w_ref[...]
    xp = jnp.dot(x, w, preferred_element_type=jnp.float32)
    xp_ref[...] = xp
    c_idx = lax.broadcasted_iota(jnp.int32, (D, H), 0)
    h_idx = lax.broadcasted_iota(jnp.int32, (D, H), 1)
    M = jnp.where((c_idx // U) == h_idx, 1.0, 0.0)
    ft = jnp.dot(xp * a1_ref[...], M, preferred_element_type=jnp.float32)
    fs = jnp.dot(xp * a2_ref[...], M, preferred_element_type=jnp.float32)
    blk = x.shape[0]
    zer = jnp.zeros((blk, D - 4 * H), jnp.float32)
    fboth_ref[...] = jnp.concatenate([ft, ft, fs, fs, zer], axis=1)


def _pre(x, w, a1, a2):
    n = x.shape[0]
    blk = 1000
    grid = n // blk
    return pl.pallas_call(
        _pre_body,
        grid=(grid,),
        in_specs=[
            pl.BlockSpec((blk, x.shape[1]), lambda i: (i, 0)),
            pl.BlockSpec((x.shape[1], D), lambda i: (0, 0)),
            pl.BlockSpec((1, D), lambda i: (0, 0)),
            pl.BlockSpec((1, D), lambda i: (0, 0)),
        ],
        out_specs=[
            pl.BlockSpec((blk, D), lambda i: (i, 0)),
            pl.BlockSpec((blk, D), lambda i: (i, 0)),
        ],
        out_shape=[
            jax.ShapeDtypeStruct((n, D), jnp.float32),
            jax.ShapeDtypeStruct((n, D), jnp.float32),
        ],
    )(x, w, a1, a2)


def _edge_body(n_pad, n_edges, k_chunk,
               xp_hbm, fboth_hbm, src_hbm, tgt_hbm,
               acc_out, ssum_out,
               tgtf, srcv, tgtv, tgt8v, catv,
               rows_g, fbuf, slotr, zb,
               acc_sh, ssum_sh, sem_a, sem_b, sem_c):
    K = k_chunk
    cid = lax.axis_index("c")
    sid = lax.axis_index("s")
    wid = sid * NC + cid
    per_w = n_edges // (NC * NS)
    n_chunks = per_w // K
    zr = zb.shape[0]
    stripe = n_pad // NS
    sstripe = n_pad // 8 // NS

    zv = jnp.zeros((LANES,), jnp.float32)
    zvi = jnp.zeros((LANES,), jnp.int32)

    def _zfill(r, _):
        for j in range(D // LANES):
            zb[r, pl.ds(LANES * j, LANES)] = zv
        return 0

    lax.fori_loop(0, zr, _zfill, 0)
    base_n = sid * stripe
    for t in range(stripe // zr):
        pltpu.sync_copy(zb, acc_sh.at[pl.ds(base_n + t * zr, zr)])
    for t in range(sstripe // zr):
        pltpu.sync_copy(zb, ssum_sh.at[pl.ds(sid * sstripe + t * zr, zr)])

    ebase = wid * per_w
    pltpu.sync_copy(tgt_hbm.at[pl.ds(ebase, per_w)],
                    tgtf.at[pl.ds(0, per_w)])
    for i in range((tgtf.shape[0] - per_w) // LANES):
        tgtf[pl.ds(per_w + LANES * i, LANES)] = zvi
    for st in (0, 16, 24):
        tgtv[pl.ds(st, LANES)] = zvi
        tgt8v[pl.ds(st, LANES)] = zvi
    plsc.subcore_barrier()

    starts = sorted({min(LANES * i, K - LANES)
                     for i in range((K + LANES - 1) // LANES)})

    def _chunk(ci, _):
        base = ci * K
        pltpu.sync_copy(src_hbm.at[pl.ds(ebase + base, K)], srcv)
        for st in starts:
            catv[pl.ds(K + st, LANES)] = srcv[pl.ds(st, LANES)]
            catv[pl.ds(st, LANES)] = tgtf[pl.ds(base + st, LANES)]
        cp_f = pltpu.async_copy(fboth_hbm.at[catv], fbuf, sem_a)
        cp_xp = pltpu.async_copy(xp_hbm.at[srcv], rows_g, sem_b)
        for st in starts:
            tv = tgtf[pl.ds(base + st, LANES)]
            tgtv[pl.ds(st, LANES)] = tv
            tgt8v[pl.ds(st, LANES)] = lax.shift_right_logical(tv, 3)
        cp_f.wait()
        cp_xp.wait()

        def _grp(t, _):
            t16 = tgtf[pl.ds(base + 8 * t, LANES)]
            for u in range(8):
                k = 8 * t + u
                vt = fbuf[k, pl.ds(0, LANES)]
                vs = fbuf[K + k, pl.ds(LANES, LANES)]
                sc = vt + vs
                sc = jnp.maximum(sc, 0.2 * sc)
                ew = jnp.exp(sc)
                uu = jnp.full((LANES,), u, jnp.int32)
                tb = t16.at[uu].get(mode="promise_in_bounds")
                slotf = jnp.bitwise_and(tb, 7).astype(jnp.float32)
                for j in range(8):
                    mf = jnp.maximum(0.0, 1.0 - jnp.abs(slotf - float(j)))
                    slotr[k, pl.ds(LANES * j, LANES)] = ew * mf
                for h in range(H):
                    hh = jnp.full((LANES,), h, jnp.int32)
                    w = ew.at[hh].get(mode="promise_in_bounds")
                    rows_g[k, pl.ds(LANES * h, LANES)] = (
                        rows_g[k, pl.ds(LANES * h, LANES)] * w)
            return 0

        lax.fori_loop(0, K // 8, _grp, 0)
        pltpu.sync_copy(slotr, ssum_sh.at[tgt8v], add=True)
        pltpu.sync_copy(rows_g, acc_sh.at[tgtv], add=True)
        return 0

    lax.fori_loop(0, n_chunks, _chunk, 0)
    plsc.subcore_barrier()

    pltpu.sync_copy(acc_sh.at[pl.ds(base_n, stripe)],
                    acc_out.at[cid, pl.ds(base_n, stripe)])
    pltpu.sync_copy(ssum_sh.at[pl.ds(sid * sstripe, sstripe)],
                    ssum_out.at[cid, pl.ds(sid * sstripe, sstripe)])


def _edge(xp, fboth, src, tgt):
    n = xp.shape[0]
    n_pad = -(-n // (NS * 64)) * (NS * 64)
    e = src.shape[0]
    K = 40
    per_w = e // (NC * NS)
    mesh = plsc.VectorSubcoreMesh(core_axis_name="c", subcore_axis_name="s",
                                  num_cores=NC, num_subcores=NS)
    f = functools.partial(
        pl.kernel,
        out_type=(
            jax.ShapeDtypeStruct((NC, n_pad, D), jnp.float32),
            jax.ShapeDtypeStruct((NC, n_pad // 8, D), jnp.float32),
        ),
        mesh=mesh,
        scratch_types=[
            pltpu.VMEM((per_w + 16,), jnp.int32),
            pltpu.VMEM((K,), jnp.int32),
            pltpu.VMEM((K,), jnp.int32),
            pltpu.VMEM((K,), jnp.int32),
            pltpu.VMEM((2 * K,), jnp.int32),
            pltpu.VMEM((K, D), jnp.float32),
            pltpu.VMEM((2 * K, D), jnp.float32),
            pltpu.VMEM((K, D), jnp.float32),
            pltpu.VMEM((K, D), jnp.float32),
            pltpu.VMEM_SHARED((n_pad, D), jnp.float32),
            pltpu.VMEM_SHARED((n_pad // 8, D), jnp.float32),
            pltpu.SemaphoreType.DMA,
            pltpu.SemaphoreType.DMA,
            pltpu.SemaphoreType.DMA,
        ],
    )(functools.partial(_edge_body, n_pad, e, K))
    return f(xp, fboth, src, tgt)


def _post_body(accp_ref, ssump_ref, bias_ref, out_ref):
    acc = accp_ref[0] + accp_ref[1]
    ssum = ssump_ref[0] + ssump_ref[1]
    j_idx = lax.broadcasted_iota(jnp.int32, (2 * H, D), 0)
    d_idx = lax.broadcasted_iota(jnp.int32, (2 * H, D), 1)
    R = jnp.where(j_idx == d_idx // U, 1.0, 0.0)
    denom = jnp.dot(ssum, R, preferred_element_type=jnp.float32)
    out = jnp.where(denom > 0.0, acc / jnp.where(denom > 0.0, denom, 1.0), 0.0)
    v = out + bias_ref[...]
    out_ref[...] = jnp.where(v > 0.0, v, jnp.exp(jnp.minimum(v, 0.0)) - 1.0)


def _post(accp, ssump, bias, n):
    blk = 1000
    grid = n // blk
    return pl.pallas_call(
        _post_body,
        grid=(grid,),
        in_specs=[
            pl.BlockSpec((NC, blk, D), lambda i: (0, i, 0)),
            pl.BlockSpec((NC, blk, 2 * H), lambda i: (0, i, 0)),
            pl.BlockSpec((D,), lambda i: (0,)),
        ],
        out_specs=pl.BlockSpec((blk, D), lambda i: (i, 0)),
        out_shape=jax.ShapeDtypeStruct((n, D), jnp.float32),
    )(accp, ssump, bias)


def kernel(x, edges, kernel, kernel_attention1, kernel_attention2, bias):
    src = edges[:, 0]
    tgt = edges[:, 1]
    a1 = kernel_attention1.reshape(1, D)
    a2 = kernel_attention2.reshape(1, D)
    xp, fboth = _pre(x, kernel, a1, a2)
    accp, ssump_packed = _edge(xp, fboth, src, tgt)
    n_pad = accp.shape[1]
    ssump = ssump_packed.reshape(NC, n_pad, 2 * H)
    return _post(accp, ssump, bias, x.shape[0])


# speedup vs baseline: 56.1982x; 1.1605x over previous
import functools

import jax
import jax.numpy as jnp
from jax import lax
from jax.experimental import pallas as pl
from jax.experimental.pallas import tpu as pltpu
from jax.experimental.pallas import tpu_sc as plsc

H = 8
U = 16
NC = 2
NS = 16
LANES = 16
D = H * U


def _pre_body(x_ref, w_ref, a1_ref, a2_ref, xp_ref, fboth_ref):
    x = x_ref[...]
    w = w_ref[...]
    xp = jnp.dot(x, w, preferred_element_type=jnp.float32)
    xp_ref[...] = xp
    c_idx = lax.broadcasted_iota(jnp.int32, (D, H), 0)
    h_idx = lax.broadcasted_iota(jnp.int32, (D, H), 1)
    M = jnp.where((c_idx // U) == h_idx, 1.0, 0.0)
    ft = jnp.dot(xp * a1_ref[...], M, preferred_element_type=jnp.float32)
    fs = jnp.dot(xp * a2_ref[...], M, preferred_element_type=jnp.float32)
    blk = x.shape[0]
    zer = jnp.zeros((blk, D - 4 * H), jnp.float32)
    fboth_ref[...] = jnp.concatenate([ft, ft, fs, fs, zer], axis=1)


def _pre(x, w, a1, a2):
    n = x.shape[0]
    blk = 1000
    grid = n // blk
    return pl.pallas_call(
        _pre_body,
        grid=(grid,),
        in_specs=[
            pl.BlockSpec((blk, x.shape[1]), lambda i: (i, 0)),
            pl.BlockSpec((x.shape[1], D), lambda i: (0, 0)),
            pl.BlockSpec((1, D), lambda i: (0, 0)),
            pl.BlockSpec((1, D), lambda i: (0, 0)),
        ],
        out_specs=[
            pl.BlockSpec((blk, D), lambda i: (i, 0)),
            pl.BlockSpec((blk, D), lambda i: (i, 0)),
        ],
        out_shape=[
            jax.ShapeDtypeStruct((n, D), jnp.float32),
            jax.ShapeDtypeStruct((n, D), jnp.float32),
        ],
    )(x, w, a1, a2)


def _edge_body(n_pad, n_edges, k_chunk,
               xp_hbm, fboth_hbm, src_hbm, tgt_hbm,
               acc_out, ssum_out,
               tgtf, srcv, tgtv, tgt8v, catv,
               rows_g, fbuf, slotr, zb,
               acc_sh, ssum_sh, sem_a, sem_b, sem_c):
    K = k_chunk
    cid = lax.axis_index("c")
    sid = lax.axis_index("s")
    wid = sid * NC + cid
    per_w = n_edges // (NC * NS)
    n_chunks = per_w // K
    zr = zb.shape[0]
    stripe = n_pad // NS
    sstripe = n_pad // 8 // NS

    zv = jnp.zeros((LANES,), jnp.float32)
    zvi = jnp.zeros((LANES,), jnp.int32)

    def _zfill(r, _):
        for j in range(D // LANES):
            zb[r, pl.ds(LANES * j, LANES)] = zv
        return 0

    lax.fori_loop(0, zr, _zfill, 0)
    base_n = sid * stripe
    for t in range(stripe // zr):
        pltpu.sync_copy(zb, acc_sh.at[pl.ds(base_n + t * zr, zr)])
    for t in range(sstripe // zr):
        pltpu.sync_copy(zb, ssum_sh.at[pl.ds(sid * sstripe + t * zr, zr)])

    ebase = wid * per_w
    pltpu.sync_copy(tgt_hbm.at[pl.ds(ebase, per_w)],
                    tgtf.at[pl.ds(0, per_w)])
    for i in range((tgtf.shape[0] - per_w) // LANES):
        tgtf[pl.ds(per_w + LANES * i, LANES)] = zvi
    for st in (0, 16, 24):
        tgtv[pl.ds(st, LANES)] = zvi
        tgt8v[pl.ds(st, LANES)] = zvi
    plsc.subcore_barrier()

    def _wait_scatters():
        pltpu.make_async_copy(rows_g, acc_sh.at[tgtv], sem_c).wait()
        pltpu.make_async_copy(slotr, ssum_sh.at[tgt8v], sem_c).wait()

    pltpu.async_copy(zb, acc_sh.at[tgtv], sem_c, add=True)
    pltpu.async_copy(zb, ssum_sh.at[tgt8v], sem_c, add=True)

    starts = sorted({min(LANES * i, K - LANES)
                     for i in range((K + LANES - 1) // LANES)})

    def _chunk(ci, _):
        base = ci * K
        pltpu.sync_copy(src_hbm.at[pl.ds(ebase + base, K)], srcv)
        for st in starts:
            catv[pl.ds(K + st, LANES)] = srcv[pl.ds(st, LANES)]
            catv[pl.ds(st, LANES)] = tgtf[pl.ds(base + st, LANES)]
        _wait_scatters()
        cp_f = pltpu.async_copy(fboth_hbm.at[catv], fbuf, sem_a)
        cp_xp = pltpu.async_copy(xp_hbm.at[srcv], rows_g, sem_b)
        for st in starts:
            tv = tgtf[pl.ds(base + st, LANES)]
            tgtv[pl.ds(st, LANES)] = tv
            tgt8v[pl.ds(st, LANES)] = lax.shift_right_logical(tv, 3)
        cp_f.wait()
        cp_xp.wait()

        def _grp(t, _):
            t16 = tgtf[pl.ds(base + 8 * t, LANES)]
            for u in range(8):
                k = 8 * t + u
                vt = fbuf[k, pl.ds(0, LANES)]
                vs = fbuf[K + k, pl.ds(LANES, LANES)]
                sc = vt + vs
                sc = jnp.maximum(sc, 0.2 * sc)
                ew = jnp.exp(sc)
                uu = jnp.full((LANES,), u, jnp.int32)
                tb = t16.at[uu].get(mode="promise_in_bounds")
                slotf = jnp.bitwise_and(tb, 7).astype(jnp.float32)
                for j in range(8):
                    mf = jnp.maximum(0.0, 1.0 - jnp.abs(slotf - float(j)))
                    slotr[k, pl.ds(LANES * j, LANES)] = ew * mf
                for h in range(H):
                    hh = jnp.full((LANES,), h, jnp.int32)
                    w = ew.at[hh].get(mode="promise_in_bounds")
                    rows_g[k, pl.ds(LANES * h, LANES)] = (
                        rows_g[k, pl.ds(LANES * h, LANES)] * w)
            return 0

        lax.fori_loop(0, K // 8, _grp, 0)
        pltpu.async_copy(slotr, ssum_sh.at[tgt8v], sem_c, add=True)
        pltpu.async_copy(rows_g, acc_sh.at[tgtv], sem_c, add=True)
        return 0

    lax.fori_loop(0, n_chunks, _chunk, 0)
    _wait_scatters()
    plsc.subcore_barrier()

    pltpu.sync_copy(acc_sh.at[pl.ds(base_n, stripe)],
                    acc_out.at[cid, pl.ds(base_n, stripe)])
    pltpu.sync_copy(ssum_sh.at[pl.ds(sid * sstripe, sstripe)],
                    ssum_out.at[cid, pl.ds(sid * sstripe, sstripe)])


def _edge(xp, fboth, src, tgt):
    n = xp.shape[0]
    n_pad = -(-n // (NS * 64)) * (NS * 64)
    e = src.shape[0]
    K = 40
    per_w = e // (NC * NS)
    mesh = plsc.VectorSubcoreMesh(core_axis_name="c", subcore_axis_name="s",
                                  num_cores=NC, num_subcores=NS)
    f = functools.partial(
        pl.kernel,
        out_type=(
            jax.ShapeDtypeStruct((NC, n_pad, D), jnp.float32),
            jax.ShapeDtypeStruct((NC, n_pad // 8, D), jnp.float32),
        ),
        mesh=mesh,
        scratch_types=[
            pltpu.VMEM((per_w + 16,), jnp.int32),
            pltpu.VMEM((K,), jnp.int32),
            pltpu.VMEM((K,), jnp.int32),
            pltpu.VMEM((K,), jnp.int32),
            pltpu.VMEM((2 * K,), jnp.int32),
            pltpu.VMEM((K, D), jnp.float32),
            pltpu.VMEM((2 * K, D), jnp.float32),
            pltpu.VMEM((K, D), jnp.float32),
            pltpu.VMEM((K, D), jnp.float32),
            pltpu.VMEM_SHARED((n_pad, D), jnp.float32),
            pltpu.VMEM_SHARED((n_pad // 8, D), jnp.float32),
            pltpu.SemaphoreType.DMA,
            pltpu.SemaphoreType.DMA,
            pltpu.SemaphoreType.DMA,
        ],
    )(functools.partial(_edge_body, n_pad, e, K))
    return f(xp, fboth, src, tgt)


def _post_body(accp_ref, ssump_ref, bias_ref, out_ref):
    acc = accp_ref[0] + accp_ref[1]
    ssum = ssump_ref[0] + ssump_ref[1]
    j_idx = lax.broadcasted_iota(jnp.int32, (2 * H, D), 0)
    d_idx = lax.broadcasted_iota(jnp.int32, (2 * H, D), 1)
    R = jnp.where(j_idx == d_idx // U, 1.0, 0.0)
    denom = jnp.dot(ssum, R, preferred_element_type=jnp.float32)
    out = jnp.where(denom > 0.0, acc / jnp.where(denom > 0.0, denom, 1.0), 0.0)
    v = out + bias_ref[...]
    out_ref[...] = jnp.where(v > 0.0, v, jnp.exp(jnp.minimum(v, 0.0)) - 1.0)


def _post(accp, ssump, bias, n):
    blk = 1000
    grid = n // blk
    return pl.pallas_call(
        _post_body,
        grid=(grid,),
        in_specs=[
            pl.BlockSpec((NC, blk, D), lambda i: (0, i, 0)),
            pl.BlockSpec((NC, blk, 2 * H), lambda i: (0, i, 0)),
            pl.BlockSpec((D,), lambda i: (0,)),
        ],
        out_specs=pl.BlockSpec((blk, D), lambda i: (i, 0)),
        out_shape=jax.ShapeDtypeStruct((n, D), jnp.float32),
    )(accp, ssump, bias)


def kernel(x, edges, kernel, kernel_attention1, kernel_attention2, bias):
    src = edges[:, 0]
    tgt = edges[:, 1]
    a1 = kernel_attention1.reshape(1, D)
    a2 = kernel_attention2.reshape(1, D)
    xp, fboth = _pre(x, kernel, a1, a2)
    accp, ssump_packed = _edge(xp, fboth, src, tgt)
    n_pad = accp.shape[1]
    ssump = ssump_packed.reshape(NC, n_pad, 2 * H)
    return _post(accp, ssump, bias, x.shape[0])


# speedup vs baseline: 56.4657x; 1.0048x over previous
import functools

import jax
import jax.numpy as jnp
from jax import lax
from jax.experimental import pallas as pl
from jax.experimental.pallas import tpu as pltpu
from jax.experimental.pallas import tpu_sc as plsc

H = 8
U = 16
NC = 2
NS = 16
LANES = 16
D = H * U


def _pre_body(x_ref, w_ref, a1_ref, a2_ref, xp_ref, fboth_ref):
    x = x_ref[...]
    w = w_ref[...]
    xp = jnp.dot(x, w, preferred_element_type=jnp.float32)
    xp_ref[...] = xp
    c_idx = lax.broadcasted_iota(jnp.int32, (D, H), 0)
    h_idx = lax.broadcasted_iota(jnp.int32, (D, H), 1)
    M = jnp.where((c_idx // U) == h_idx, 1.0, 0.0)
    ft = jnp.dot(xp * a1_ref[...], M, preferred_element_type=jnp.float32)
    fs = jnp.dot(xp * a2_ref[...], M, preferred_element_type=jnp.float32)
    blk = x.shape[0]
    zer = jnp.zeros((blk, D - 4 * H), jnp.float32)
    fboth_ref[...] = jnp.concatenate([ft, ft, fs, fs, zer], axis=1)


def _pre(x, w, a1, a2):
    n = x.shape[0]
    blk = 1000
    grid = n // blk
    return pl.pallas_call(
        _pre_body,
        grid=(grid,),
        in_specs=[
            pl.BlockSpec((blk, x.shape[1]), lambda i: (i, 0)),
            pl.BlockSpec((x.shape[1], D), lambda i: (0, 0)),
            pl.BlockSpec((1, D), lambda i: (0, 0)),
            pl.BlockSpec((1, D), lambda i: (0, 0)),
        ],
        out_specs=[
            pl.BlockSpec((blk, D), lambda i: (i, 0)),
            pl.BlockSpec((blk, D), lambda i: (i, 0)),
        ],
        out_shape=[
            jax.ShapeDtypeStruct((n, D), jnp.float32),
            jax.ShapeDtypeStruct((n, D), jnp.float32),
        ],
    )(x, w, a1, a2)


def _edge_body(n_pad, n_edges, k_chunk,
               xp_hbm, fboth_hbm, src_hbm, tgt_hbm,
               acc_out, ssum_out,
               tgtf, srcv, tgtv, tgt8v, catv,
               rows_g, fbuf, slotr, zb,
               acc_sh, ssum_sh, sem_a, sem_b, sem_c):
    K = k_chunk
    cid = lax.axis_index("c")
    sid = lax.axis_index("s")
    wid = sid * NC + cid
    per_w = n_edges // (NC * NS)
    n_chunks = per_w // K
    zr = zb.shape[0]
    stripe = n_pad // NS
    sstripe = n_pad // 8 // NS

    zv = jnp.zeros((LANES,), jnp.float32)
    zvi = jnp.zeros((LANES,), jnp.int32)

    def _zfill(r, _):
        for j in range(D // LANES):
            zb[r, pl.ds(LANES * j, LANES)] = zv
        return 0

    lax.fori_loop(0, zr, _zfill, 0)
    base_n = sid * stripe
    for t in range(stripe // zr):
        pltpu.sync_copy(zb, acc_sh.at[pl.ds(base_n + t * zr, zr)])
    for t in range(sstripe // zr):
        pltpu.sync_copy(zb, ssum_sh.at[pl.ds(sid * sstripe + t * zr, zr)])

    ebase = wid * per_w
    pltpu.sync_copy(tgt_hbm.at[pl.ds(ebase, per_w)],
                    tgtf.at[pl.ds(0, per_w)])
    for i in range((tgtf.shape[0] - per_w) // LANES):
        tgtf[pl.ds(per_w + LANES * i, LANES)] = zvi
    for st in (0, 16, 24):
        tgtv[pl.ds(st, LANES)] = zvi
        tgt8v[pl.ds(st, LANES)] = zvi
    plsc.subcore_barrier()

    def _wait_scatters():
        pltpu.make_async_copy(rows_g, acc_sh.at[tgtv], sem_c).wait()
        pltpu.make_async_copy(slotr, ssum_sh.at[tgt8v], sem_c).wait()

    pltpu.async_copy(zb, acc_sh.at[tgtv], sem_c, add=True)
    pltpu.async_copy(zb, ssum_sh.at[tgt8v], sem_c, add=True)

    starts = sorted({min(LANES * i, K - LANES)
                     for i in range((K + LANES - 1) // LANES)})

    def _chunk(ci, _):
        base = ci * K
        pltpu.sync_copy(src_hbm.at[pl.ds(ebase + base, K)], srcv)
        for st in starts:
            catv[pl.ds(K + st, LANES)] = srcv[pl.ds(st, LANES)]
            catv[pl.ds(st, LANES)] = tgtf[pl.ds(base + st, LANES)]
        cp_f = pltpu.async_copy(fboth_hbm.at[catv], fbuf, sem_a)
        _wait_scatters()
        cp_xp = pltpu.async_copy(xp_hbm.at[srcv], rows_g, sem_b)
        for st in starts:
            tv = tgtf[pl.ds(base + st, LANES)]
            tgtv[pl.ds(st, LANES)] = tv
            tgt8v[pl.ds(st, LANES)] = lax.shift_right_logical(tv, 3)
        cp_f.wait()
        cp_xp.wait()

        def _grp(t, _):
            t16 = tgtf[pl.ds(base + 8 * t, LANES)]
            for u in range(8):
                k = 8 * t + u
                vt = fbuf[k, pl.ds(0, LANES)]
                vs = fbuf[K + k, pl.ds(LANES, LANES)]
                sc = vt + vs
                sc = jnp.maximum(sc, 0.2 * sc)
                ew = jnp.exp(sc)
                uu = jnp.full((LANES,), u, jnp.int32)
                tb = t16.at[uu].get(mode="promise_in_bounds")
                slotf = jnp.bitwise_and(tb, 7).astype(jnp.float32)
                for j in range(8):
                    mf = jnp.maximum(0.0, 1.0 - jnp.abs(slotf - float(j)))
                    slotr[k, pl.ds(LANES * j, LANES)] = ew * mf
                for h in range(H):
                    hh = jnp.full((LANES,), h, jnp.int32)
                    w = ew.at[hh].get(mode="promise_in_bounds")
                    rows_g[k, pl.ds(LANES * h, LANES)] = (
                        rows_g[k, pl.ds(LANES * h, LANES)] * w)
            return 0

        lax.fori_loop(0, K // 8, _grp, 0)
        pltpu.async_copy(slotr, ssum_sh.at[tgt8v], sem_c, add=True)
        pltpu.async_copy(rows_g, acc_sh.at[tgtv], sem_c, add=True)
        return 0

    lax.fori_loop(0, n_chunks, _chunk, 0)
    _wait_scatters()
    plsc.subcore_barrier()

    pltpu.sync_copy(acc_sh.at[pl.ds(base_n, stripe)],
                    acc_out.at[cid, pl.ds(base_n, stripe)])
    pltpu.sync_copy(ssum_sh.at[pl.ds(sid * sstripe, sstripe)],
                    ssum_out.at[cid, pl.ds(sid * sstripe, sstripe)])


def _edge(xp, fboth, src, tgt):
    n = xp.shape[0]
    n_pad = -(-n // (NS * 64)) * (NS * 64)
    e = src.shape[0]
    K = 40
    per_w = e // (NC * NS)
    mesh = plsc.VectorSubcoreMesh(core_axis_name="c", subcore_axis_name="s",
                                  num_cores=NC, num_subcores=NS)
    f = functools.partial(
        pl.kernel,
        out_type=(
            jax.ShapeDtypeStruct((NC, n_pad, D), jnp.float32),
            jax.ShapeDtypeStruct((NC, n_pad // 8, D), jnp.float32),
        ),
        mesh=mesh,
        scratch_types=[
            pltpu.VMEM((per_w + 16,), jnp.int32),
            pltpu.VMEM((K,), jnp.int32),
            pltpu.VMEM((K,), jnp.int32),
            pltpu.VMEM((K,), jnp.int32),
            pltpu.VMEM((2 * K,), jnp.int32),
            pltpu.VMEM((K, D), jnp.float32),
            pltpu.VMEM((2 * K, D), jnp.float32),
            pltpu.VMEM((K, D), jnp.float32),
            pltpu.VMEM((K, D), jnp.float32),
            pltpu.VMEM_SHARED((n_pad, D), jnp.float32),
            pltpu.VMEM_SHARED((n_pad // 8, D), jnp.float32),
            pltpu.SemaphoreType.DMA,
            pltpu.SemaphoreType.DMA,
            pltpu.SemaphoreType.DMA,
        ],
    )(functools.partial(_edge_body, n_pad, e, K))
    return f(xp, fboth, src, tgt)


def _post_body(accp_ref, ssump_ref, bias_ref, out_ref):
    acc = accp_ref[0] + accp_ref[1]
    ssum = ssump_ref[0] + ssump_ref[1]
    j_idx = lax.broadcasted_iota(jnp.int32, (2 * H, D), 0)
    d_idx = lax.broadcasted_iota(jnp.int32, (2 * H, D), 1)
    R = jnp.where(j_idx == d_idx // U, 1.0, 0.0)
    denom = jnp.dot(ssum, R, preferred_element_type=jnp.float32)
    out = jnp.where(denom > 0.0, acc / jnp.where(denom > 0.0, denom, 1.0), 0.0)
    v = out + bias_ref[...]
    out_ref[...] = jnp.where(v > 0.0, v, jnp.exp(jnp.minimum(v, 0.0)) - 1.0)


def _post(accp, ssump, bias, n):
    blk = 1000
    grid = n // blk
    return pl.pallas_call(
        _post_body,
        grid=(grid,),
        in_specs=[
            pl.BlockSpec((NC, blk, D), lambda i: (0, i, 0)),
            pl.BlockSpec((NC, blk, 2 * H), lambda i: (0, i, 0)),
            pl.BlockSpec((D,), lambda i: (0,)),
        ],
        out_specs=pl.BlockSpec((blk, D), lambda i: (i, 0)),
        out_shape=jax.ShapeDtypeStruct((n, D), jnp.float32),
    )(accp, ssump, bias)


def kernel(x, edges, kernel, kernel_attention1, kernel_attention2, bias):
    src = edges[:, 0]
    tgt = edges[:, 1]
    a1 = kernel_attention1.reshape(1, D)
    a2 = kernel_attention2.reshape(1, D)
    xp, fboth = _pre(x, kernel, a1, a2)
    accp, ssump_packed = _edge(xp, fboth, src, tgt)
    n_pad = accp.shape[1]
    ssump = ssump_packed.reshape(NC, n_pad, 2 * H)
    return _post(accp, ssump, bias, x.shape[0])


# speedup vs baseline: 62.3854x; 1.1048x over previous
import functools

import jax
import jax.numpy as jnp
from jax import lax
from jax.experimental import pallas as pl
from jax.experimental.pallas import tpu as pltpu
from jax.experimental.pallas import tpu_sc as plsc

H = 8
U = 16
NC = 2
NS = 16
LANES = 16
D = H * U


def _pre_body(x_ref, w_ref, a1_ref, a2_ref, xp_ref, fboth_ref):
    x = x_ref[...]
    w = w_ref[...]
    xp = jnp.dot(x, w, preferred_element_type=jnp.float32)
    xp_ref[...] = xp
    c_idx = lax.broadcasted_iota(jnp.int32, (D, H), 0)
    h_idx = lax.broadcasted_iota(jnp.int32, (D, H), 1)
    M = jnp.where((c_idx // U) == h_idx, 1.0, 0.0)
    ft = jnp.dot(xp * a1_ref[...], M, preferred_element_type=jnp.float32)
    fs = jnp.dot(xp * a2_ref[...], M, preferred_element_type=jnp.float32)
    blk = x.shape[0]
    zer = jnp.zeros((blk, D - 4 * H), jnp.float32)
    fboth_ref[...] = jnp.concatenate([ft, ft, fs, fs, zer], axis=1)


def _pre(x, w, a1, a2):
    n = x.shape[0]
    blk = 1000
    grid = n // blk
    return pl.pallas_call(
        _pre_body,
        grid=(grid,),
        in_specs=[
            pl.BlockSpec((blk, x.shape[1]), lambda i: (i, 0)),
            pl.BlockSpec((x.shape[1], D), lambda i: (0, 0)),
            pl.BlockSpec((1, D), lambda i: (0, 0)),
            pl.BlockSpec((1, D), lambda i: (0, 0)),
        ],
        out_specs=[
            pl.BlockSpec((blk, D), lambda i: (i, 0)),
            pl.BlockSpec((blk, D), lambda i: (i, 0)),
        ],
        out_shape=[
            jax.ShapeDtypeStruct((n, D), jnp.float32),
            jax.ShapeDtypeStruct((n, D), jnp.float32),
        ],
    )(x, w, a1, a2)


def _edge_body(n_pad, n_edges, k_chunk,
               xp_hbm, fboth_hbm, src_hbm, tgt_hbm,
               acc_out, ssum_out,
               tgtf, srcv, srcv2, tgtv, tgt8v, catv,
               rows_g, fbuf, slotr, zb,
               acc_sh, ssum_sh, sem_a, sem_b, sem_c, sem_d):
    K = k_chunk
    cid = lax.axis_index("c")
    sid = lax.axis_index("s")
    wid = sid * NC + cid
    per_w = n_edges // (NC * NS)
    n_chunks = per_w // K
    zr = zb.shape[0]
    stripe = n_pad // NS
    sstripe = n_pad // 8 // NS

    zv = jnp.zeros((LANES,), jnp.float32)
    zvi = jnp.zeros((LANES,), jnp.int32)

    def _zfill(r, _):
        for j in range(D // LANES):
            zb[r, pl.ds(LANES * j, LANES)] = zv
        return 0

    lax.fori_loop(0, zr, _zfill, 0)
    base_n = sid * stripe
    for t in range(stripe // zr):
        pltpu.sync_copy(zb, acc_sh.at[pl.ds(base_n + t * zr, zr)])
    for t in range(sstripe // zr):
        pltpu.sync_copy(zb, ssum_sh.at[pl.ds(sid * sstripe + t * zr, zr)])

    ebase = pl.multiple_of(wid * per_w, 8)
    pltpu.sync_copy(tgt_hbm.at[pl.ds(ebase, per_w)],
                    tgtf.at[pl.ds(0, per_w)])
    for i in range((tgtf.shape[0] - per_w) // LANES):
        tgtf[pl.ds(per_w + LANES * i, LANES)] = zvi
    for st in (0, 16, 24):
        tgtv[pl.ds(st, LANES)] = zvi
        tgt8v[pl.ds(st, LANES)] = zvi
    plsc.subcore_barrier()

    def _wait_scatters():
        pltpu.make_async_copy(rows_g, acc_sh.at[tgtv], sem_c).wait()
        pltpu.make_async_copy(slotr, ssum_sh.at[tgt8v], sem_c).wait()

    pltpu.async_copy(zb, acc_sh.at[tgtv], sem_c, add=True)
    pltpu.async_copy(zb, ssum_sh.at[tgt8v], sem_c, add=True)
    pltpu.async_copy(src_hbm.at[pl.ds(pl.multiple_of(ebase, 8), K)],
                     srcv2, sem_d)

    starts = sorted({min(LANES * i, K - LANES)
                     for i in range((K + LANES - 1) // LANES)})

    def _chunk(ci, _):
        base = pl.multiple_of(ci * K, 8)
        pltpu.make_async_copy(
            src_hbm.at[pl.ds(pl.multiple_of(ebase, 8), K)],
            srcv2, sem_d).wait()
        for st in starts:
            sv = srcv2[pl.ds(st, LANES)]
            srcv[pl.ds(st, LANES)] = sv
            catv[pl.ds(K + st, LANES)] = sv
            catv[pl.ds(st, LANES)] = tgtf[pl.ds(base + st, LANES)]
        pltpu.async_copy(
            src_hbm.at[pl.ds(pl.multiple_of(ebase + base + K, 8), K)],
            srcv2, sem_d)
        cp_f = pltpu.async_copy(fboth_hbm.at[catv], fbuf, sem_a)
        _wait_scatters()
        cp_xp = pltpu.async_copy(xp_hbm.at[srcv], rows_g, sem_b)
        for st in starts:
            tv = tgtf[pl.ds(base + st, LANES)]
            tgtv[pl.ds(st, LANES)] = tv
            tgt8v[pl.ds(st, LANES)] = lax.shift_right_logical(tv, 3)
        cp_f.wait()
        cp_xp.wait()

        def _grp(t, _):
            t16 = tgtf[pl.ds(base + 8 * t, LANES)]
            for u in range(8):
                k = 8 * t + u
                vt = fbuf[k, pl.ds(0, LANES)]
                vs = fbuf[K + k, pl.ds(LANES, LANES)]
                sc = vt + vs
                sc = jnp.maximum(sc, 0.2 * sc)
                ew = jnp.exp(sc)
                uu = jnp.full((LANES,), u, jnp.int32)
                tb = t16.at[uu].get(mode="promise_in_bounds")
                slotf = jnp.bitwise_and(tb, 7).astype(jnp.float32)
                for j in range(8):
                    mf = jnp.maximum(0.0, 1.0 - jnp.abs(slotf - float(j)))
                    slotr[k, pl.ds(LANES * j, LANES)] = ew * mf
                for h in range(H):
                    hh = jnp.full((LANES,), h, jnp.int32)
                    w = ew.at[hh].get(mode="promise_in_bounds")
                    rows_g[k, pl.ds(LANES * h, LANES)] = (
                        rows_g[k, pl.ds(LANES * h, LANES)] * w)
            return 0

        lax.fori_loop(0, K // 8, _grp, 0)
        pltpu.async_copy(slotr, ssum_sh.at[tgt8v], sem_c, add=True)
        pltpu.async_copy(rows_g, acc_sh.at[tgtv], sem_c, add=True)
        return 0

    lax.fori_loop(0, n_chunks, _chunk, 0)
    pltpu.make_async_copy(
        src_hbm.at[pl.ds(pl.multiple_of(ebase, 8), K)], srcv2, sem_d).wait()
    _wait_scatters()
    plsc.subcore_barrier()

    pltpu.sync_copy(acc_sh.at[pl.ds(base_n, stripe)],
                    acc_out.at[cid, pl.ds(base_n, stripe)])
    pltpu.sync_copy(ssum_sh.at[pl.ds(sid * sstripe, sstripe)],
                    ssum_out.at[cid, pl.ds(sid * sstripe, sstripe)])


def _edge(xp, fboth, src, tgt):
    n = xp.shape[0]
    n_pad = -(-n // (NS * 64)) * (NS * 64)
    e = tgt.shape[0]
    K = 40
    per_w = e // (NC * NS)
    mesh = plsc.VectorSubcoreMesh(core_axis_name="c", subcore_axis_name="s",
                                  num_cores=NC, num_subcores=NS)
    f = functools.partial(
        pl.kernel,
        out_type=(
            jax.ShapeDtypeStruct((NC, n_pad, D), jnp.float32),
            jax.ShapeDtypeStruct((NC, n_pad // 8, D), jnp.float32),
        ),
        mesh=mesh,
        scratch_types=[
            pltpu.VMEM((per_w + 16,), jnp.int32),
            pltpu.VMEM((K,), jnp.int32),
            pltpu.VMEM((K,), jnp.int32),
            pltpu.VMEM((K,), jnp.int32),
            pltpu.VMEM((K,), jnp.int32),
            pltpu.VMEM((2 * K,), jnp.int32),
            pltpu.VMEM((K, D), jnp.float32),
            pltpu.VMEM((2 * K, D), jnp.float32),
            pltpu.VMEM((K, D), jnp.float32),
            pltpu.VMEM((K, D), jnp.float32),
            pltpu.VMEM_SHARED((n_pad, D), jnp.float32),
            pltpu.VMEM_SHARED((n_pad // 8, D), jnp.float32),
            pltpu.SemaphoreType.DMA,
            pltpu.SemaphoreType.DMA,
            pltpu.SemaphoreType.DMA,
            pltpu.SemaphoreType.DMA,
        ],
    )(functools.partial(_edge_body, n_pad, e, K))
    return f(xp, fboth, src, tgt)


def _post_body(accp_ref, ssump_ref, bias_ref, out_ref):
    acc = accp_ref[0] + accp_ref[1]
    ssum = ssump_ref[0] + ssump_ref[1]
    j_idx = lax.broadcasted_iota(jnp.int32, (2 * H, D), 0)
    d_idx = lax.broadcasted_iota(jnp.int32, (2 * H, D), 1)
    R = jnp.where(j_idx == d_idx // U, 1.0, 0.0)
    denom = jnp.dot(ssum, R, preferred_element_type=jnp.float32)
    out = jnp.where(denom > 0.0, acc / jnp.where(denom > 0.0, denom, 1.0), 0.0)
    v = out + bias_ref[...]
    out_ref[...] = jnp.where(v > 0.0, v, jnp.exp(jnp.minimum(v, 0.0)) - 1.0)


def _post(accp, ssump, bias, n):
    blk = 1000
    grid = n // blk
    return pl.pallas_call(
        _post_body,
        grid=(grid,),
        in_specs=[
            pl.BlockSpec((NC, blk, D), lambda i: (0, i, 0)),
            pl.BlockSpec((NC, blk, 2 * H), lambda i: (0, i, 0)),
            pl.BlockSpec((D,), lambda i: (0,)),
        ],
        out_specs=pl.BlockSpec((blk, D), lambda i: (i, 0)),
        out_shape=jax.ShapeDtypeStruct((n, D), jnp.float32),
    )(accp, ssump, bias)


def kernel(x, edges, kernel, kernel_attention1, kernel_attention2, bias):
    src = edges[:, 0]
    tgt = edges[:, 1]
    a1 = kernel_attention1.reshape(1, D)
    a2 = kernel_attention2.reshape(1, D)
    xp, fboth = _pre(x, kernel, a1, a2)
    srcp = jnp.concatenate([src, jnp.zeros((40,), jnp.int32)])
    accp, ssump_packed = _edge(xp, fboth, srcp, tgt)
    n_pad = accp.shape[1]
    ssump = ssump_packed.reshape(NC, n_pad, 2 * H)
    return _post(accp, ssump, bias, x.shape[0])
